# baseline (speedup 1.0000x reference)
import functools

import jax
import jax.numpy as jnp
from jax.experimental import pallas as pl
from jax.experimental.pallas import tpu as pltpu

_CH = 200
_NB = 5


def _body(comp_ref, adj_hbm, seq_ref, w_ref, out_hbm,
          buf, stage, adj_sems, out_sems):
    n = seq_ref.shape[0]
    nc = n // _CH
    wv = w_ref[...] * comp_ref[0, 0]

    def adj_copy(c, j):
        return pltpu.make_async_copy(
            adj_hbm.at[pl.ds(c * _CH, _CH), :], buf.at[j], adj_sems.at[j])

    def out_copy(c, j):
        return pltpu.make_async_copy(
            stage.at[j], out_hbm.at[pl.ds(c * _CH, _CH), :], out_sems.at[j])

    for j in range(_NB):
        adj_copy(j, j).start()

    def group(g, carry):
        for j in range(_NB):
            c = g * _NB + j
            adj_copy(c, j).wait()
            gmat = jnp.dot(buf[j], seq_ref[...],
                           preferred_element_type=jnp.float32)
            r = jnp.maximum(jnp.dot(gmat, wv,
                                    preferred_element_type=jnp.float32), 0.0)

            @pl.when(g > 0)
            def _wait_prev_out():
                out_copy(c - _NB, j).wait()

            stage[j] = r
            out_copy(c, j).start()

            @pl.when(c + _NB < nc)
            def _next_adj():
                adj_copy(c + _NB, j).start()
        return carry

    jax.lax.fori_loop(0, nc // _NB, group, 0)

    for j in range(_NB):
        out_copy(nc - _NB + j, j).wait()


def kernel(seqs, adjs, comp, weight):
    seq = seqs[0]
    adj = adjs[0]
    w = weight[0]
    n, in_ft = seq.shape
    out_ft = w.shape[1]

    out = pl.pallas_call(
        _body,
        in_specs=[
            pl.BlockSpec(memory_space=pltpu.SMEM),
            pl.BlockSpec(memory_space=pl.ANY),
            pl.BlockSpec((n, in_ft), lambda: (0, 0)),
            pl.BlockSpec((in_ft, out_ft), lambda: (0, 0)),
        ],
        out_specs=pl.BlockSpec(memory_space=pl.ANY),
        out_shape=jax.ShapeDtypeStruct((n, out_ft), jnp.float32),
        scratch_shapes=[
            pltpu.VMEM((_NB, _CH, n), jnp.float32),
            pltpu.VMEM((_NB, _CH, out_ft), jnp.float32),
            pltpu.SemaphoreType.DMA((_NB,)),
            pltpu.SemaphoreType.DMA((_NB,)),
        ],
    )(comp, adj, seq, w)
    return out
